# Initial kernel scaffold; baseline (speedup 1.0000x reference)
import jax
import jax.numpy as jnp
from jax.experimental import pallas as pl


def kernel(x, edge_index, node_type, edge_type, W_l, b_l, W_r):
    raise NotImplementedError("write your pallas kernel here")


# speedup vs baseline: 4.2586x; 4.2586x over previous
import functools

import jax
import jax.numpy as jnp
from jax import lax
from jax.experimental import pallas as pl
from jax.experimental.pallas import tpu as pltpu
from jax.experimental.pallas import tpu_sc as plsc

N = 10000
E = 320000
D = 128

NC, NS = 2, 16
NW = NC * NS
CHUNK = 128
EPW = E // NW
HCHUNK = 16
NPASS = 5
NCHUNK = HCHUNK * NPASS
EPW_PAD = NCHUNK * CHUNK
ROWS_ACC = 10112
RPT = ROWS_ACC // NS
DUMMY = N
CW = 16


def _sc_body(x_hbm, src_hbm, dst_hbm,
             sum_out, cnt_out,
             acc_sum, acc_cnt, idx_src, idx_dst, rowbuf, ones_v, cntbuf, gsem):
    c = lax.axis_index("c")
    s = lax.axis_index("s")
    w = c * NS + s
    r0 = s * RPT

    zrow = jnp.zeros((16,), jnp.float32)

    @pl.loop(0, CHUNK)
    def _fill(i):
        for k in range(D // 16):
            rowbuf[i, pl.ds(k * 16, 16)] = zrow
        ones_v[i, :] = jnp.ones((16,), jnp.float32)
        cntbuf[i, :] = zrow

    for k in range(4):
        pltpu.sync_copy(rowbuf, acc_sum.at[pl.ds(r0 + k * CHUNK, CHUNK)])
        pltpu.sync_copy(cntbuf, acc_cnt.at[pl.ds(r0 + k * CHUNK, CHUNK)])
    pltpu.sync_copy(rowbuf.at[pl.ds(0, RPT - 4 * CHUNK)],
                    acc_sum.at[pl.ds(r0 + 4 * CHUNK, RPT - 4 * CHUNK)])
    pltpu.sync_copy(cntbuf.at[pl.ds(0, RPT - 4 * CHUNK)],
                    acc_cnt.at[pl.ds(r0 + 4 * CHUNK, RPT - 4 * CHUNK)])
    plsc.subcore_barrier()

    for h in range(NPASS):
        pltpu.sync_copy(src_hbm.at[w, pl.ds(h * HCHUNK, HCHUNK)], idx_src)
        pltpu.sync_copy(dst_hbm.at[w, pl.ds(h * HCHUNK, HCHUNK)], idx_dst)

        @pl.loop(0, HCHUNK)
        def _chunk(j):
            pltpu.async_copy(x_hbm.at[idx_src.at[j]], rowbuf, gsem).wait()
            pltpu.sync_copy(rowbuf, acc_sum.at[idx_dst.at[j]], add=True)
            pltpu.sync_copy(ones_v, acc_cnt.at[idx_dst.at[j]], add=True)

    plsc.subcore_barrier()
    for k in range(4):
        pltpu.sync_copy(acc_sum.at[pl.ds(r0 + k * CHUNK, CHUNK)], rowbuf)
        pltpu.sync_copy(rowbuf, sum_out.at[c, pl.ds(r0 + k * CHUNK, CHUNK)])
        pltpu.sync_copy(acc_cnt.at[pl.ds(r0 + k * CHUNK, CHUNK)], cntbuf)
        pltpu.sync_copy(cntbuf, cnt_out.at[c, pl.ds(r0 + k * CHUNK, CHUNK)])
    tail = RPT - 4 * CHUNK
    pltpu.sync_copy(acc_sum.at[pl.ds(r0 + 4 * CHUNK, tail)],
                    rowbuf.at[pl.ds(0, tail)])
    pltpu.sync_copy(rowbuf.at[pl.ds(0, tail)],
                    sum_out.at[c, pl.ds(r0 + 4 * CHUNK, tail)])
    pltpu.sync_copy(acc_cnt.at[pl.ds(r0 + 4 * CHUNK, tail)],
                    cntbuf.at[pl.ds(0, tail)])
    pltpu.sync_copy(cntbuf.at[pl.ds(0, tail)],
                    cnt_out.at[c, pl.ds(r0 + 4 * CHUNK, tail)])


_sc_scatter = functools.partial(
    pl.kernel,
    out_type=[
        jax.ShapeDtypeStruct((NC, ROWS_ACC, D), jnp.float32),
        jax.ShapeDtypeStruct((NC, ROWS_ACC, CW), jnp.float32),
    ],
    mesh=plsc.VectorSubcoreMesh(core_axis_name="c", subcore_axis_name="s"),
    scratch_types=[
        pltpu.VMEM_SHARED((ROWS_ACC, D), jnp.float32),
        pltpu.VMEM_SHARED((ROWS_ACC, CW), jnp.float32),
        pltpu.VMEM((HCHUNK, CHUNK), jnp.int32),
        pltpu.VMEM((HCHUNK, CHUNK), jnp.int32),
        pltpu.VMEM((CHUNK, D), jnp.float32),
        pltpu.VMEM((CHUNK, CW), jnp.float32),
        pltpu.VMEM((CHUNK, CW), jnp.float32),
        pltpu.SemaphoreType.DMA,
    ],
    compiler_params=pltpu.CompilerParams(use_tc_tiling_on_sc=False),
)(_sc_body)


def _tc_body(x_ref, s0_ref, s1_ref, c0_ref, c1_ref, wl_ref, wr_ref, b_ref,
             o_ref):
    cnt = c0_ref[:, 0:1] + c1_ref[:, 0:1]
    agg = (s0_ref[...] + s1_ref[...]) / jnp.maximum(cnt, 1.0)
    dn = (((1,), (1,)), ((), ()))
    o_ref[...] = (
        lax.dot_general(agg, wl_ref[...], dn, preferred_element_type=jnp.float32)
        + lax.dot_general(x_ref[...], wr_ref[...], dn,
                          preferred_element_type=jnp.float32)
        + b_ref[...])


def _tc_dense(x, s0, s1, c0, c1, W_l, b_l, W_r):
    blk = 1000
    grid = N // blk
    row_spec = pl.BlockSpec((blk, D), lambda i: (i, 0))
    cnt_spec = pl.BlockSpec((blk, CW), lambda i: (i, 0))
    full = pl.BlockSpec((D, D), lambda i: (0, 0))
    bias = pl.BlockSpec((1, D), lambda i: (0, 0))
    return pl.pallas_call(
        _tc_body,
        grid=(grid,),
        in_specs=[row_spec, row_spec, row_spec, cnt_spec, cnt_spec,
                  full, full, bias],
        out_specs=row_spec,
        out_shape=jax.ShapeDtypeStruct((N, D), jnp.float32),
    )(x, s0, s1, c0, c1, W_l, W_r, b_l.reshape(1, D))


def kernel(x, edge_index, node_type, edge_type, W_l, b_l, W_r):
    src = edge_index[0].reshape(NW, EPW)
    dst = edge_index[1].reshape(NW, EPW)
    pad = EPW_PAD - EPW
    src_p = jnp.concatenate(
        [src, jnp.zeros((NW, pad), jnp.int32)], axis=1).reshape(NW, NCHUNK, CHUNK)
    dst_p = jnp.concatenate(
        [dst, jnp.full((NW, pad), DUMMY, jnp.int32)], axis=1).reshape(NW, NCHUNK, CHUNK)
    sums, cnts = _sc_scatter(x, src_p, dst_p)
    return _tc_dense(x, sums[0, :N], sums[1, :N], cnts[0, :N], cnts[1, :N],
                     W_l, b_l, W_r)


# speedup vs baseline: 4.4697x; 1.0496x over previous
import functools

import jax
import jax.numpy as jnp
from jax import lax
from jax.experimental import pallas as pl
from jax.experimental.pallas import tpu as pltpu
from jax.experimental.pallas import tpu_sc as plsc

N = 10000
E = 320000
D = 128
DA = 144

NC, NS = 2, 16
NW = NC * NS
CHUNK = 64
EPW = E // NW
PCHUNK = 8
NPASS = 20
NCHUNK = PCHUNK * NPASS
EPW_PAD = NCHUNK * CHUNK
ROWS_ACC = 10112
RPT = ROWS_ACC // NS
DUMMY = N
DEPTH = 4


def _sc_body(x_hbm, src_hbm, dst_hbm, sum_out,
             acc, idx_src0, idx_src1, idx_dst0, idx_dst1,
             rb0, rb1, rb2, rb3,
             gs0, gs1, gs2, gs3, ss0, ss1, ss2, ss3):
    c = lax.axis_index("c")
    s = lax.axis_index("s")
    w = c * NS + s
    r0 = s * RPT
    idx_src = (idx_src0, idx_src1)
    idx_dst = (idx_dst0, idx_dst1)
    rowbuf = (rb0, rb1, rb2, rb3)
    gsem = (gs0, gs1, gs2, gs3)
    ssem = (ss0, ss1, ss2, ss3)

    zrow = jnp.zeros((16,), jnp.float32)

    @pl.loop(0, CHUNK)
    def _fill(i):
        for k in range(DA // 16):
            rb0[i, pl.ds(k * 16, 16)] = zrow

    for k in range(9):
        pltpu.sync_copy(rb0, acc.at[pl.ds(r0 + k * CHUNK, CHUNK)])
    pltpu.sync_copy(rb0.at[pl.ds(0, RPT - 9 * CHUNK)],
                    acc.at[pl.ds(r0 + 9 * CHUNK, RPT - 9 * CHUNK)])
    plsc.subcore_barrier()

    g_desc = [None] * DEPTH
    s_desc = [None] * DEPTH
    pending = [None] * NCHUNK
    t = 0
    for p in range(NPASS):
        hs, hd = idx_src[p % 2], idx_dst[p % 2]
        pltpu.sync_copy(src_hbm.at[w, pl.ds(p * PCHUNK, PCHUNK)], hs)
        pltpu.sync_copy(dst_hbm.at[w, pl.ds(p * PCHUNK, PCHUNK)], hd)
        for j in range(PCHUNK):
            b = t % DEPTH
            if s_desc[b] is not None:
                s_desc[b].wait()
            g_desc[b] = pltpu.async_copy(
                x_hbm.at[hs.at[j]], rowbuf[b], gsem[b])
            pending[t] = (b, hd.at[j])
            tp = t - 2
            if tp >= 0:
                pb, prow = pending[tp]
                g_desc[pb].wait()
                s_desc[pb] = pltpu.async_copy(
                    rowbuf[pb], acc.at[prow], ssem[pb], add=True)
            t += 1
    for tp in (NCHUNK - 2, NCHUNK - 1):
        pb, prow = pending[tp]
        g_desc[pb].wait()
        s_desc[pb] = pltpu.async_copy(
            rowbuf[pb], acc.at[prow], ssem[pb], add=True)
    for b in range(DEPTH):
        if s_desc[b] is not None:
            s_desc[b].wait()

    plsc.subcore_barrier()
    pltpu.sync_copy(acc.at[pl.ds(r0, RPT)], sum_out.at[c, pl.ds(r0, RPT)])


_sc_scatter = functools.partial(
    pl.kernel,
    out_type=[
        jax.ShapeDtypeStruct((NC, ROWS_ACC, DA), jnp.float32),
    ],
    mesh=plsc.VectorSubcoreMesh(core_axis_name="c", subcore_axis_name="s"),
    scratch_types=(
        [pltpu.VMEM_SHARED((ROWS_ACC, DA), jnp.float32)]
        + [pltpu.VMEM((PCHUNK, CHUNK), jnp.int32)] * 4
        + [pltpu.VMEM((CHUNK, DA), jnp.float32)] * DEPTH
        + [pltpu.SemaphoreType.DMA] * (2 * DEPTH)
    ),
    compiler_params=pltpu.CompilerParams(use_tc_tiling_on_sc=False),
)(_sc_body)


def _tc_body(x_ref, s0_ref, s1_ref, wl_ref, wr_ref, b_ref, o_ref):
    cnt = s0_ref[:, D:D + 1] + s1_ref[:, D:D + 1]
    agg = (s0_ref[:, :D] + s1_ref[:, :D]) / jnp.maximum(cnt, 1.0)
    dn = (((1,), (1,)), ((), ()))
    o_ref[...] = (
        lax.dot_general(agg, wl_ref[...], dn, preferred_element_type=jnp.float32)
        + lax.dot_general(x_ref[...], wr_ref[...], dn,
                          preferred_element_type=jnp.float32)
        + b_ref[...])


def _tc_dense(x, s0, s1, W_l, b_l, W_r):
    blk = 1000
    grid = N // blk
    row_spec = pl.BlockSpec((blk, D), lambda i: (i, 0))
    aug_spec = pl.BlockSpec((blk, DA), lambda i: (i, 0))
    full = pl.BlockSpec((D, D), lambda i: (0, 0))
    bias = pl.BlockSpec((1, D), lambda i: (0, 0))
    return pl.pallas_call(
        _tc_body,
        grid=(grid,),
        in_specs=[row_spec, aug_spec, aug_spec, full, full, bias],
        out_specs=row_spec,
        out_shape=jax.ShapeDtypeStruct((N, D), jnp.float32),
    )(x, s0, s1, W_l, W_r, b_l.reshape(1, D))


def kernel(x, edge_index, node_type, edge_type, W_l, b_l, W_r):
    x_aug = jnp.concatenate([x, jnp.ones((N, DA - D), jnp.float32)], axis=1)
    src = edge_index[0].reshape(NW, EPW)
    dst = edge_index[1].reshape(NW, EPW)
    pad = EPW_PAD - EPW
    src_p = jnp.concatenate(
        [src, jnp.zeros((NW, pad), jnp.int32)], axis=1).reshape(NW, NCHUNK, CHUNK)
    dst_p = jnp.concatenate(
        [dst, jnp.full((NW, pad), DUMMY, jnp.int32)], axis=1).reshape(NW, NCHUNK, CHUNK)
    (sums,) = _sc_scatter(x_aug, src_p, dst_p)
    return _tc_dense(x, sums[0, :N], sums[1, :N], W_l, b_l, W_r)
